# Initial kernel scaffold; baseline (speedup 1.0000x reference)
import functools

import jax
import jax.numpy as jnp
from jax.experimental import pallas as pl

E = 8
F_IN = 128
H1 = 64
H2 = 64
F_OUT = 16


def _celu(x):
    return jnp.where(x > 0, x, jnp.expm1(x))


def _mlp_block_kernel(el_ref, x_ref, w1_ref, b1_ref, w2_ref, b2_ref, w3_ref, b3_ref, o_ref):
    x = x_ref[...]
    el = el_ref[0, 0, :]
    xb = x.astype(jnp.bfloat16)
    acc = jnp.zeros((x.shape[0], F_OUT), dtype=jnp.float32)
    for e in range(E):
        w1 = w1_ref[e].astype(jnp.bfloat16)
        h = jax.lax.dot_general(xb, w1, (((1,), (1,)), ((), ())),
                                preferred_element_type=jnp.float32)
        h = _celu(h + b1_ref[e][None, :]).astype(jnp.bfloat16)
        w2 = w2_ref[e].astype(jnp.bfloat16)
        h = jax.lax.dot_general(h, w2, (((1,), (1,)), ((), ())),
                                preferred_element_type=jnp.float32)
        h = _celu(h + b2_ref[e][None, :]).astype(jnp.bfloat16)
        w3 = w3_ref[e].astype(jnp.bfloat16)
        o = jax.lax.dot_general(h, w3, (((1,), (1,)), ((), ())),
                                preferred_element_type=jnp.float32)
        o = o + b3_ref[e][None, :]
        acc = jnp.where((el == e)[:, None], o, acc)
    o_ref[...] = acc


def kernel(elements, features, W1, b1, W2, b2, W3, b3):
    n, M, f = features.shape
    N = n * M
    B = 1024
    nblk = N // B
    x = features.reshape(N, f)
    el3 = elements.reshape(nblk, 1, B)

    grid_spec = pl.GridSpec(
        grid=(nblk,),
        in_specs=[
            pl.BlockSpec((1, 1, B), lambda i: (i, 0, 0)),
            pl.BlockSpec((B, F_IN), lambda i: (i, 0)),
            pl.BlockSpec((E, H1, F_IN), lambda i: (0, 0, 0)),
            pl.BlockSpec((E, H1), lambda i: (0, 0)),
            pl.BlockSpec((E, H2, H1), lambda i: (0, 0, 0)),
            pl.BlockSpec((E, H2), lambda i: (0, 0)),
            pl.BlockSpec((E, F_OUT, H2), lambda i: (0, 0, 0)),
            pl.BlockSpec((E, F_OUT), lambda i: (0, 0)),
        ],
        out_specs=pl.BlockSpec((B, F_OUT), lambda i: (i, 0)),
    )
    y = pl.pallas_call(
        _mlp_block_kernel,
        grid_spec=grid_spec,
        out_shape=jax.ShapeDtypeStruct((N, F_OUT), jnp.float32),
    )(el3, x, W1, b1, W2, b2, W3, b3)
    return (elements, y.reshape(n, M, F_OUT))


# speedup vs baseline: 1.4096x; 1.4096x over previous
import functools

import jax
import jax.numpy as jnp
from jax.experimental import pallas as pl

E = 8
F_IN = 128
H1 = 64
H2 = 64
F_OUT = 16


def _celu(x):
    return jnp.where(x > 0, x, jnp.exp(jnp.minimum(x, 0.0)) - 1.0)


def _mlp_block_kernel(el_ref, x_ref, w1_ref, b1_ref, w2_ref, b2_ref, w3_ref, b3_ref, o_ref):
    x = x_ref[...]
    el = el_ref[...]
    xb = x.astype(jnp.bfloat16)
    acc = jnp.zeros((x.shape[0], F_OUT), dtype=jnp.float32)
    for e in range(E):
        w1 = w1_ref[e].astype(jnp.bfloat16)
        h = jax.lax.dot_general(xb, w1, (((1,), (1,)), ((), ())),
                                preferred_element_type=jnp.float32)
        h = _celu(h + b1_ref[e][None, :]).astype(jnp.bfloat16)
        w2 = w2_ref[e].astype(jnp.bfloat16)
        h = jax.lax.dot_general(h, w2, (((1,), (1,)), ((), ())),
                                preferred_element_type=jnp.float32)
        h = _celu(h + b2_ref[e][None, :]).astype(jnp.bfloat16)
        w3 = w3_ref[e].astype(jnp.bfloat16)
        o = jax.lax.dot_general(h, w3, (((1,), (1,)), ((), ())),
                                preferred_element_type=jnp.float32)
        o = o + b3_ref[e][None, :]
        acc = jnp.where(el == e, o, acc)
    o_ref[...] = acc


def kernel(elements, features, W1, b1, W2, b2, W3, b3):
    n, M, f = features.shape
    N = n * M
    B = 1024
    nblk = N // B
    x = features.reshape(N, f)
    el3 = elements.reshape(N, 1)

    grid_spec = pl.GridSpec(
        grid=(nblk,),
        in_specs=[
            pl.BlockSpec((B, 1), lambda i: (i, 0)),
            pl.BlockSpec((B, F_IN), lambda i: (i, 0)),
            pl.BlockSpec((E, H1, F_IN), lambda i: (0, 0, 0)),
            pl.BlockSpec((E, H1), lambda i: (0, 0)),
            pl.BlockSpec((E, H2, H1), lambda i: (0, 0, 0)),
            pl.BlockSpec((E, H2), lambda i: (0, 0)),
            pl.BlockSpec((E, F_OUT, H2), lambda i: (0, 0, 0)),
            pl.BlockSpec((E, F_OUT), lambda i: (0, 0)),
        ],
        out_specs=pl.BlockSpec((B, F_OUT), lambda i: (i, 0)),
    )
    y = pl.pallas_call(
        _mlp_block_kernel,
        grid_spec=grid_spec,
        out_shape=jax.ShapeDtypeStruct((N, F_OUT), jnp.float32),
    )(el3, x, W1, b1, W2, b2, W3, b3)
    return (elements, y.reshape(n, M, F_OUT))


# speedup vs baseline: 1.7541x; 1.2444x over previous
import functools

import jax
import jax.numpy as jnp
from jax.experimental import pallas as pl

E = 8
F_IN = 128
H1 = 64
H2 = 64
F_OUT = 16


def _celu(x):
    return jnp.where(x > 0, x, jnp.exp(jnp.minimum(x, 0.0)) - 1.0)


def _mlp_block_kernel(el_ref, x_ref, w1_ref, b1_ref, w2_ref, b2_ref, w3_ref, b3_ref, o_ref):
    x = x_ref[...]
    el = el_ref[...]
    xb = x.astype(jnp.bfloat16)
    B = x.shape[0]

    def layer(h, w_ref, b_ref, width):
        z = jnp.zeros((B, width), dtype=jnp.float32)
        for e in range(E):
            w = w_ref[e].astype(jnp.bfloat16)
            ze = jax.lax.dot_general(h, w, (((1,), (1,)), ((), ())),
                                     preferred_element_type=jnp.float32)
            z = jnp.where(el == e, ze + b_ref[e][None, :], z)
        return z

    h = _celu(layer(xb, w1_ref, b1_ref, H1)).astype(jnp.bfloat16)
    h = _celu(layer(h, w2_ref, b2_ref, H2)).astype(jnp.bfloat16)
    o_ref[...] = layer(h, w3_ref, b3_ref, F_OUT)


def kernel(elements, features, W1, b1, W2, b2, W3, b3):
    n, M, f = features.shape
    N = n * M
    B = 1024
    nblk = N // B
    x = features.reshape(N, f)
    el3 = elements.reshape(N, 1)

    grid_spec = pl.GridSpec(
        grid=(nblk,),
        in_specs=[
            pl.BlockSpec((B, 1), lambda i: (i, 0)),
            pl.BlockSpec((B, F_IN), lambda i: (i, 0)),
            pl.BlockSpec((E, H1, F_IN), lambda i: (0, 0, 0)),
            pl.BlockSpec((E, H1), lambda i: (0, 0)),
            pl.BlockSpec((E, H2, H1), lambda i: (0, 0, 0)),
            pl.BlockSpec((E, H2), lambda i: (0, 0)),
            pl.BlockSpec((E, F_OUT, H2), lambda i: (0, 0, 0)),
            pl.BlockSpec((E, F_OUT), lambda i: (0, 0)),
        ],
        out_specs=pl.BlockSpec((B, F_OUT), lambda i: (i, 0)),
    )
    y = pl.pallas_call(
        _mlp_block_kernel,
        grid_spec=grid_spec,
        out_shape=jax.ShapeDtypeStruct((N, F_OUT), jnp.float32),
    )(el3, x, W1, b1, W2, b2, W3, b3)
    return (elements, y.reshape(n, M, F_OUT))
